# Initial kernel scaffold; baseline (speedup 1.0000x reference)
import functools

import jax
import jax.numpy as jnp
from jax import lax
from jax.experimental import pallas as pl
from jax.experimental.pallas import tpu as pltpu
from jax.experimental.pallas import tpu_sc as plsc

N_NODES = 10000
N_EDGES = 320000
D_IN = 128
DA = 144
NPAD = 10240
NC = 2
NS = 16
NW = NC * NS
CHUNK = 128
CPT = -(-N_EDGES // (NW * CHUNK))
EPAD = NW * CPT * CHUNK
ROWS_PER_TILE = NPAD // NS

_mesh = plsc.VectorSubcoreMesh(core_axis_name="c", subcore_axis_name="s")


@functools.partial(
    pl.kernel,
    out_type=jax.ShapeDtypeStruct((NC, NPAD, DA), jnp.float32),
    mesh=_mesh,
    scratch_types=[
        pltpu.VMEM((CHUNK,), jnp.int32),
        pltpu.VMEM((CHUNK,), jnp.int32),
        pltpu.VMEM((CHUNK, DA), jnp.float32),
        pltpu.VMEM_SHARED((NPAD, DA), jnp.float32),
        pltpu.SemaphoreType.DMA,
    ],
)
def _sc_spmm(table_h, src_h, dst_h, zeros_h, out_h,
             src_v, dst_v, rows_v, acc_sh, sem):
    c = lax.axis_index("c")
    s = lax.axis_index("s")
    wid = s * NC + c

    pltpu.sync_copy(zeros_h, acc_sh.at[pl.ds(s * ROWS_PER_TILE, ROWS_PER_TILE)])
    plsc.subcore_barrier()

    base = wid * CPT

    def chunk_body(i, carry):
        row = base + i
        pltpu.sync_copy(src_h.at[row], src_v)
        pltpu.sync_copy(dst_h.at[row], dst_v)
        pltpu.async_copy(table_h.at[src_v], rows_v, sem).wait()
        pltpu.sync_copy(rows_v, acc_sh.at[dst_v], add=True)
        return carry

    lax.fori_loop(0, CPT, chunk_body, 0)
    plsc.subcore_barrier()

    pltpu.sync_copy(
        acc_sh.at[pl.ds(s * ROWS_PER_TILE, ROWS_PER_TILE)],
        out_h.at[c, pl.ds(s * ROWS_PER_TILE, ROWS_PER_TILE)],
    )


_ROWS_BLK = 256
_N_BLKS = NPAD // _ROWS_BLK


def _add_body(a_ref, b_ref, o_ref):
    o_ref[...] = a_ref[...] + b_ref[...]


def _tc_add(a, b):
    return pl.pallas_call(
        _add_body,
        grid=(_N_BLKS,),
        in_specs=[
            pl.BlockSpec((_ROWS_BLK, DA), lambda i: (i, 0)),
            pl.BlockSpec((_ROWS_BLK, DA), lambda i: (i, 0)),
        ],
        out_specs=pl.BlockSpec((_ROWS_BLK, DA), lambda i: (i, 0)),
        out_shape=jax.ShapeDtypeStruct((NPAD, DA), jnp.float32),
    )(a, b)


def _final_body(xa_ref, y1_ref, p2a_ref, p2b_ref, w_ref, o_ref):
    y2 = p2a_ref[...] + p2b_ref[...]
    o_ref[:, 0:D_IN] = jnp.dot(xa_ref[...], w_ref[0],
                               preferred_element_type=jnp.float32)
    o_ref[:, D_IN:2 * D_IN] = jnp.dot(y1_ref[...], w_ref[1],
                                      preferred_element_type=jnp.float32)
    o_ref[:, 2 * D_IN:3 * D_IN] = jnp.dot(y2, w_ref[2],
                                          preferred_element_type=jnp.float32)


def _tc_final(xa, y1, p2a, p2b, w_all):
    return pl.pallas_call(
        _final_body,
        grid=(_N_BLKS,),
        in_specs=[
            pl.BlockSpec((_ROWS_BLK, DA), lambda i: (i, 0)),
            pl.BlockSpec((_ROWS_BLK, DA), lambda i: (i, 0)),
            pl.BlockSpec((_ROWS_BLK, DA), lambda i: (i, 0)),
            pl.BlockSpec((_ROWS_BLK, DA), lambda i: (i, 0)),
            pl.BlockSpec((3, DA, D_IN), lambda i: (0, 0, 0)),
        ],
        out_specs=pl.BlockSpec((_ROWS_BLK, 3 * D_IN), lambda i: (i, 0)),
        out_shape=jax.ShapeDtypeStruct((NPAD, 3 * D_IN), jnp.float32),
    )(xa, y1, p2a, p2b, w_all)


def kernel(x, edge_index, W0, b0, W1, b1, W2, b2):
    x = x.astype(jnp.float32)

    xa = jnp.zeros((NPAD, DA), jnp.float32)
    xa = xa.at[:N_NODES, :D_IN].set(x)
    xa = xa.at[:N_NODES, D_IN].set(1.0)

    src = edge_index[0].astype(jnp.int32)
    dst = edge_index[1].astype(jnp.int32)
    pad = jnp.full((EPAD - N_EDGES,), N_NODES, jnp.int32)
    src2d = jnp.concatenate([src, pad]).reshape(NW * CPT, CHUNK)
    dst2d = jnp.concatenate([dst, pad]).reshape(NW * CPT, CHUNK)

    zeros = jnp.zeros((ROWS_PER_TILE, DA), jnp.float32)

    def wa(W, b):
        return jnp.concatenate(
            [W.T.astype(jnp.float32), b.astype(jnp.float32)[None, :],
             jnp.zeros((DA - D_IN - 1, D_IN), jnp.float32)], axis=0)

    w_all = jnp.stack([wa(W0, b0), wa(W1, b1), wa(W2, b2)])

    p1 = _sc_spmm(xa, src2d, dst2d, zeros)
    y1 = _tc_add(p1[0], p1[1])
    p2 = _sc_spmm(y1, src2d, dst2d, zeros)
    out = _tc_final(xa, y1, p2[0], p2[1], w_all)
    return out[:N_NODES]


# speedup vs baseline: 2.4002x; 2.4002x over previous
import functools

import jax
import jax.numpy as jnp
from jax import lax
from jax.experimental import pallas as pl
from jax.experimental.pallas import tpu as pltpu
from jax.experimental.pallas import tpu_sc as plsc

N_NODES = 10000
N_EDGES = 320000
D = 128
NPAD = 10240
NC = 2
NS = 16
NW = NC * NS
CHUNK = 128
CPT = -(-N_EDGES // (NW * CHUNK))
EPAD = NW * CPT * CHUNK
ROWS_PER_TILE = NPAD // NS

_mesh = plsc.VectorSubcoreMesh(core_axis_name="c", subcore_axis_name="s")


@functools.partial(
    pl.kernel,
    out_type=jax.ShapeDtypeStruct((NC, NPAD, D), jnp.float32),
    mesh=_mesh,
    scratch_types=[
        pltpu.VMEM((CHUNK,), jnp.int32),
        pltpu.VMEM((CHUNK,), jnp.int32),
        pltpu.VMEM((CHUNK, D), jnp.float32),
        pltpu.VMEM_SHARED((NPAD, D), jnp.float32),
        pltpu.SemaphoreType.DMA,
    ],
)
def _sc_spmm(table_h, src_h, dst_h, zeros_h, out_h,
             src_v, dst_v, rows_v, acc_sh, sem):
    c = lax.axis_index("c")
    s = lax.axis_index("s")
    wid = s * NC + c

    pltpu.sync_copy(zeros_h, acc_sh.at[pl.ds(s * ROWS_PER_TILE, ROWS_PER_TILE)])
    plsc.subcore_barrier()

    base = wid * CPT

    def chunk_body(i, carry):
        row = base + i
        pltpu.sync_copy(src_h.at[row], src_v)
        pltpu.sync_copy(dst_h.at[row], dst_v)
        pltpu.async_copy(table_h.at[src_v], rows_v, sem).wait()
        pltpu.sync_copy(rows_v, acc_sh.at[dst_v], add=True)
        return carry

    lax.fori_loop(0, CPT, chunk_body, 0)
    plsc.subcore_barrier()

    pltpu.sync_copy(
        acc_sh.at[pl.ds(s * ROWS_PER_TILE, ROWS_PER_TILE)],
        out_h.at[c, pl.ds(s * ROWS_PER_TILE, ROWS_PER_TILE)],
    )


_ROWS_BLK = 256
_N_BLKS = NPAD // _ROWS_BLK


def _pre_body(xa_ref, w_ref, b_ref, x0_ref, h1_ref, h2_ref):
    xa = xa_ref[...]
    x0_ref[...] = jnp.dot(xa, w_ref[0], preferred_element_type=jnp.float32) + b_ref[0, 0]
    h1_ref[...] = jnp.dot(xa, w_ref[1], preferred_element_type=jnp.float32) + b_ref[0, 1]
    h2_ref[...] = jnp.dot(xa, w_ref[2], preferred_element_type=jnp.float32) + b_ref[0, 2]


def _tc_pre(xa, w_all, b_all):
    shp = jax.ShapeDtypeStruct((NPAD, D), jnp.float32)
    return pl.pallas_call(
        _pre_body,
        grid=(_N_BLKS,),
        in_specs=[
            pl.BlockSpec((_ROWS_BLK, D), lambda i: (i, 0)),
            pl.BlockSpec((3, D, D), lambda i: (0, 0, 0)),
            pl.BlockSpec((1, 3, D), lambda i: (0, 0, 0)),
        ],
        out_specs=[pl.BlockSpec((_ROWS_BLK, D), lambda i: (i, 0))] * 3,
        out_shape=[shp, shp, shp],
    )(xa, w_all, b_all)


def _add_body(a_ref, o_ref):
    o_ref[...] = a_ref[0] + a_ref[1]


def _tc_add(p):
    return pl.pallas_call(
        _add_body,
        grid=(_N_BLKS,),
        in_specs=[pl.BlockSpec((2, _ROWS_BLK, D), lambda i: (0, i, 0))],
        out_specs=pl.BlockSpec((_ROWS_BLK, D), lambda i: (i, 0)),
        out_shape=jax.ShapeDtypeStruct((NPAD, D), jnp.float32),
    )(p)


def _final_body(x0_ref, p1_ref, p3_ref, o_ref):
    o_ref[:, 0:D] = x0_ref[...]
    o_ref[:, D:2 * D] = p1_ref[0] + p1_ref[1]
    o_ref[:, 2 * D:3 * D] = p3_ref[0] + p3_ref[1]


def _tc_final(x0, p1, p3):
    return pl.pallas_call(
        _final_body,
        grid=(_N_BLKS,),
        in_specs=[
            pl.BlockSpec((_ROWS_BLK, D), lambda i: (i, 0)),
            pl.BlockSpec((2, _ROWS_BLK, D), lambda i: (0, i, 0)),
            pl.BlockSpec((2, _ROWS_BLK, D), lambda i: (0, i, 0)),
        ],
        out_specs=pl.BlockSpec((_ROWS_BLK, 3 * D), lambda i: (i, 0)),
        out_shape=jax.ShapeDtypeStruct((NPAD, 3 * D), jnp.float32),
    )(x0, p1, p3)


def kernel(x, edge_index, W0, b0, W1, b1, W2, b2):
    x = x.astype(jnp.float32)

    xa = jnp.zeros((NPAD, D), jnp.float32)
    xa = xa.at[:N_NODES].set(x)

    src = edge_index[0].astype(jnp.int32)
    dst = edge_index[1].astype(jnp.int32)
    pad = jnp.full((EPAD - N_EDGES,), N_NODES, jnp.int32)
    src2d = jnp.concatenate([src, pad]).reshape(NW * CPT, CHUNK)
    dst2d = jnp.concatenate([dst, pad]).reshape(NW * CPT, CHUNK)

    zeros = jnp.zeros((ROWS_PER_TILE, D), jnp.float32)

    w_all = jnp.stack([W0.T, W1.T, W2.T]).astype(jnp.float32)
    b_all = jnp.stack([b0, b1, b2]).astype(jnp.float32)[None]

    x0, h1, h2 = _tc_pre(xa, w_all, b_all)
    p1 = _sc_spmm(h1, src2d, dst2d, zeros)
    p2 = _sc_spmm(h2, src2d, dst2d, zeros)
    y2 = _tc_add(p2)
    p3 = _sc_spmm(y2, src2d, dst2d, zeros)
    out = _tc_final(x0, p1, p3)
    return out[:N_NODES]
